# baseline (device time: 29161 ns/iter reference)
import jax
import jax.numpy as jnp
from jax import lax
from jax.experimental import pallas as pl
from jax.experimental.pallas import tpu as pltpu

NCHUNK = 4


def kernel(x, assign, W1, W2):
    t, d = x.shape
    e_loc, _, f = W1.shape
    rows = t // NCHUNK

    assign2d = assign.reshape(t, 1)

    def body(x_hbm, a_hbm, w1_hbm, w2_hbm, out_ref,
             xv, av, w1v, w2v, xsend, xrecv, arecv, prt, rbuf,
             ldma_sems, send_sems, recv_sems):
        my_x = lax.axis_index("x")
        my_y = lax.axis_index("y")
        my_z = lax.axis_index("z")
        peer = (my_x, 1 - my_y, my_z)

        cp_x = pltpu.make_async_copy(x_hbm, xv, ldma_sems.at[0])
        cp_x.start()
        cp_a = pltpu.make_async_copy(a_hbm, av, ldma_sems.at[1])
        cp_a.start()
        cp_w = []
        for le in range(e_loc):
            c1 = pltpu.make_async_copy(w1_hbm.at[le], w1v.at[le],
                                       ldma_sems.at[2 + 2 * le])
            c1.start()
            c2 = pltpu.make_async_copy(w2_hbm.at[le], w2v.at[le],
                                       ldma_sems.at[3 + 2 * le])
            c2.start()
            cp_w.append((c1, c2))

        barrier_sem = pltpu.get_barrier_semaphore()
        pl.semaphore_signal(barrier_sem, inc=1, device_id=peer,
                            device_id_type=pl.DeviceIdType.MESH)
        pl.semaphore_wait(barrier_sem, 1)

        cp_x.wait()
        cp_a.wait()
        xsend[...] = xv[...].astype(jnp.bfloat16)
        rdma_x = pltpu.make_async_remote_copy(
            src_ref=xsend, dst_ref=xrecv,
            send_sem=send_sems.at[0], recv_sem=recv_sems.at[0],
            device_id=peer, device_id_type=pl.DeviceIdType.MESH)
        rdma_x.start()
        rdma_a = pltpu.make_async_remote_copy(
            src_ref=av, dst_ref=arecv,
            send_sem=send_sems.at[1], recv_sem=recv_sems.at[1],
            device_id=peer, device_id_type=pl.DeviceIdType.MESH)
        rdma_a.start()

        def expert_partial(le, Xb, A):
            e_glob = e_loc * my_y + le
            h = jnp.maximum(
                jnp.dot(Xb, w1v[le].astype(jnp.bfloat16),
                        preferred_element_type=jnp.float32),
                0.0).astype(jnp.bfloat16)
            o = jnp.dot(h, w2v[le].astype(jnp.bfloat16),
                        preferred_element_type=jnp.float32)
            return jnp.where(A == e_glob, o, 0.0)

        acc_m = jnp.zeros((t, d), jnp.float32)
        for le in range(e_loc):
            cp_w[le][0].wait()
            cp_w[le][1].wait()
            acc_m = acc_m + expert_partial(le, xsend[...], av[...])

        rdma_x.wait()
        rdma_a.wait()

        sends = []
        for c in range(NCHUNK):
            sl = pl.ds(c * rows, rows)
            acc_p = jnp.zeros((rows, d), jnp.float32)
            for le in range(e_loc):
                acc_p = acc_p + expert_partial(le, xrecv[sl, :], arecv[sl, :])
            prt[sl, :] = acc_p.astype(jnp.bfloat16)
            rdma_p = pltpu.make_async_remote_copy(
                src_ref=prt.at[sl], dst_ref=rbuf.at[sl],
                send_sem=send_sems.at[2 + c], recv_sem=recv_sems.at[2 + c],
                device_id=peer, device_id_type=pl.DeviceIdType.MESH)
            rdma_p.start()
            sends.append(rdma_p)

        for c in range(NCHUNK):
            sl = pl.ds(c * rows, rows)
            sends[c].wait_recv()
            out_ref[sl, :] = (acc_m[c * rows:(c + 1) * rows, :]
                              + rbuf[sl, :].astype(jnp.float32))
        for c in range(NCHUNK):
            sends[c].wait_send()

    return pl.pallas_call(
        body,
        out_shape=jax.ShapeDtypeStruct((t, d), jnp.float32),
        in_specs=[
            pl.BlockSpec(memory_space=pl.ANY),
            pl.BlockSpec(memory_space=pl.ANY),
            pl.BlockSpec(memory_space=pl.ANY),
            pl.BlockSpec(memory_space=pl.ANY),
        ],
        out_specs=pl.BlockSpec(memory_space=pltpu.VMEM),
        scratch_shapes=[
            pltpu.VMEM((t, d), jnp.float32),
            pltpu.VMEM((t, 1), jnp.int32),
            pltpu.VMEM((e_loc, d, f), jnp.float32),
            pltpu.VMEM((e_loc, f, d), jnp.float32),
            pltpu.VMEM((t, d), jnp.bfloat16),
            pltpu.VMEM((t, d), jnp.bfloat16),
            pltpu.VMEM((t, 1), jnp.int32),
            pltpu.VMEM((t, d), jnp.bfloat16),
            pltpu.VMEM((t, d), jnp.bfloat16),
            pltpu.SemaphoreType.DMA((2 + 2 * e_loc,)),
            pltpu.SemaphoreType.DMA((2 + NCHUNK,)),
            pltpu.SemaphoreType.DMA((2 + NCHUNK,)),
        ],
        compiler_params=pltpu.CompilerParams(collective_id=0),
    )(x, assign2d, W1, W2)


# device time: 25191 ns/iter; 1.1576x vs baseline; 1.1576x over previous
import jax
import jax.numpy as jnp
from jax import lax
from jax.experimental import pallas as pl
from jax.experimental.pallas import tpu as pltpu

NCHUNK = 4


def kernel(x, assign, W1, W2):
    t, d = x.shape
    e_loc, _, f = W1.shape
    rows = t // NCHUNK

    assign2d = assign.reshape(t, 1)

    def body(x_ref, a_ref, w1_ref, w2_ref, out_ref,
             xsend, xrecv, arecv, prt, rbuf, send_sems, recv_sems):
        my_x = lax.axis_index("x")
        my_y = lax.axis_index("y")
        my_z = lax.axis_index("z")
        peer = (my_x, 1 - my_y, my_z)

        barrier_sem = pltpu.get_barrier_semaphore()
        pl.semaphore_signal(barrier_sem, inc=1, device_id=peer,
                            device_id_type=pl.DeviceIdType.MESH)
        pl.semaphore_wait(barrier_sem, 1)

        rdma_a = pltpu.make_async_remote_copy(
            src_ref=a_ref, dst_ref=arecv,
            send_sem=send_sems.at[0], recv_sem=recv_sems.at[0],
            device_id=peer, device_id_type=pl.DeviceIdType.MESH)
        rdma_a.start()
        xsend[...] = x_ref[...].astype(jnp.bfloat16)
        xsends = []
        for c in range(NCHUNK):
            sl = pl.ds(c * rows, rows)
            r = pltpu.make_async_remote_copy(
                src_ref=xsend.at[sl], dst_ref=xrecv.at[sl],
                send_sem=send_sems.at[1 + c], recv_sem=recv_sems.at[1 + c],
                device_id=peer, device_id_type=pl.DeviceIdType.MESH)
            r.start()
            xsends.append(r)

        def moe(Xb, A):
            n = Xb.shape[0]
            acc = jnp.zeros((n, d), jnp.float32)
            for le in range(e_loc):
                e_glob = e_loc * my_y + le
                h = jnp.maximum(
                    jnp.dot(Xb, w1_ref[le].astype(jnp.bfloat16),
                            preferred_element_type=jnp.float32),
                    0.0).astype(jnp.bfloat16)
                o = jnp.dot(h, w2_ref[le].astype(jnp.bfloat16),
                            preferred_element_type=jnp.float32)
                acc = acc + jnp.where(A == e_glob, o, 0.0)
            return acc

        acc_m = moe(xsend[...], a_ref[...])

        rdma_a.wait_recv()
        psends = []
        for c in range(NCHUNK):
            sl = pl.ds(c * rows, rows)
            xsends[c].wait_recv()
            acc_p = moe(xrecv[sl, :], arecv[sl, :])
            prt[sl, :] = acc_p.astype(jnp.bfloat16)
            r = pltpu.make_async_remote_copy(
                src_ref=prt.at[sl], dst_ref=rbuf.at[sl],
                send_sem=send_sems.at[1 + NCHUNK + c],
                recv_sem=recv_sems.at[1 + NCHUNK + c],
                device_id=peer, device_id_type=pl.DeviceIdType.MESH)
            r.start()
            psends.append(r)

        for c in range(NCHUNK):
            sl = pl.ds(c * rows, rows)
            psends[c].wait_recv()
            out_ref[sl, :] = (acc_m[c * rows:(c + 1) * rows, :]
                              + rbuf[sl, :].astype(jnp.float32))

        rdma_a.wait_send()
        for r in xsends:
            r.wait_send()
        for r in psends:
            r.wait_send()

    nsem = 1 + 2 * NCHUNK
    return pl.pallas_call(
        body,
        out_shape=jax.ShapeDtypeStruct((t, d), jnp.float32),
        in_specs=[
            pl.BlockSpec(memory_space=pltpu.VMEM),
            pl.BlockSpec(memory_space=pltpu.VMEM),
            pl.BlockSpec(memory_space=pltpu.VMEM),
            pl.BlockSpec(memory_space=pltpu.VMEM),
        ],
        out_specs=pl.BlockSpec(memory_space=pltpu.VMEM),
        scratch_shapes=[
            pltpu.VMEM((t, d), jnp.bfloat16),
            pltpu.VMEM((t, d), jnp.bfloat16),
            pltpu.VMEM((t, 1), jnp.int32),
            pltpu.VMEM((t, d), jnp.bfloat16),
            pltpu.VMEM((t, d), jnp.bfloat16),
            pltpu.SemaphoreType.DMA((nsem,)),
            pltpu.SemaphoreType.DMA((nsem,)),
        ],
        compiler_params=pltpu.CompilerParams(collective_id=0),
    )(x, assign2d, W1, W2)
